# baseline (speedup 1.0000x reference)
import functools

import jax
import jax.numpy as jnp
from jax import lax
from jax.experimental import pallas as pl
from jax.experimental.pallas import tpu as pltpu
from jax.experimental.pallas import tpu_sc as plsc

_NC = 2
_NS = 16
_NW = _NC * _NS
_K = 16


@functools.lru_cache(maxsize=None)
def _build(B, D):
    assert B % (8 * _NW) == 0
    b_per_w = B // _NW
    mesh = plsc.VectorSubcoreMesh(core_axis_name="c", subcore_axis_name="s")

    @functools.partial(
        pl.kernel,
        mesh=mesh,
        out_type=(
            jax.ShapeDtypeStruct((B, D), jnp.float32),
            jax.ShapeDtypeStruct((B, D), jnp.float32),
            jax.ShapeDtypeStruct((B, 1), jnp.float32),
            jax.ShapeDtypeStruct((B, 1), jnp.float32),
        ),
        scratch_types=[
            pltpu.VMEM((b_per_w,), jnp.int32),
            pltpu.VMEM((b_per_w,), jnp.int32),
            pltpu.SemaphoreType.DMA,
        ],
        compiler_params=pltpu.CompilerParams(use_tc_tiling_on_sc=True),
    )
    def glove_gather(ctr_hbm, cxt_hbm, ctr_tab, cxt_tab, ctr_bias, cxt_bias,
                     ctr_out, cxt_out, ctr_b_out, cxt_b_out,
                     ctr_idx, cxt_idx, sem):
        wid = lax.axis_index("s") * _NC + lax.axis_index("c")
        base = wid * b_per_w
        pltpu.sync_copy(ctr_hbm.at[pl.ds(base, b_per_w)], ctr_idx)
        pltpu.sync_copy(cxt_hbm.at[pl.ds(base, b_per_w)], cxt_idx)

        def body(g, carry):
            del carry
            vc = ctr_idx[pl.ds(g * _K, _K)]
            vx = cxt_idx[pl.ds(g * _K, _K)]
            copies = []
            for j in range(_K):
                row = base + g * _K + j
                ic = vc[j]
                ix = vx[j]
                copies.append(pltpu.async_copy(
                    ctr_tab.at[pl.ds(ic, 1)], ctr_out.at[pl.ds(row, 1)], sem))
                copies.append(pltpu.async_copy(
                    cxt_tab.at[pl.ds(ix, 1)], cxt_out.at[pl.ds(row, 1)], sem))
                copies.append(pltpu.async_copy(
                    ctr_bias.at[pl.ds(ic, 1)], ctr_b_out.at[pl.ds(row, 1)], sem))
                copies.append(pltpu.async_copy(
                    cxt_bias.at[pl.ds(ix, 1)], cxt_b_out.at[pl.ds(row, 1)], sem))
            for c in copies:
                c.wait()
            return 0

        lax.fori_loop(0, b_per_w // _K, body, 0)

    return glove_gather


def kernel(ctr, cxt, ctr_table, cxt_table, ctr_bias_table, cxt_bias_table):
    B = ctr.shape[0]
    D = ctr_table.shape[1]
    fn = _build(B, D)
    return fn(
        ctr.astype(jnp.int32),
        cxt.astype(jnp.int32),
        ctr_table,
        cxt_table,
        ctr_bias_table,
        cxt_bias_table,
    )


# speedup vs baseline: 1.8865x; 1.8865x over previous
import functools

import jax
import jax.numpy as jnp
from jax import lax
from jax.experimental import pallas as pl
from jax.experimental.pallas import tpu as pltpu
from jax.experimental.pallas import tpu_sc as plsc

_NC = 2
_NS = 16
_NW = _NC * _NS
_L = 16


@functools.lru_cache(maxsize=None)
def _build(B, D, V):
    assert B % (8 * _NW) == 0
    b_per_w = B // _NW
    mesh = plsc.VectorSubcoreMesh(core_axis_name="c", subcore_axis_name="s")

    @functools.partial(
        pl.kernel,
        mesh=mesh,
        out_type=(
            jax.ShapeDtypeStruct((B, D), jnp.float32),
            jax.ShapeDtypeStruct((B, D), jnp.float32),
            jax.ShapeDtypeStruct((B,), jnp.float32),
            jax.ShapeDtypeStruct((B,), jnp.float32),
        ),
        scratch_types=[
            pltpu.VMEM((b_per_w,), jnp.int32),
            pltpu.VMEM((b_per_w,), jnp.int32),
            pltpu.VMEM((b_per_w, D), jnp.float32),
            pltpu.VMEM((b_per_w, D), jnp.float32),
            pltpu.VMEM((b_per_w,), jnp.float32),
            pltpu.VMEM((b_per_w,), jnp.float32),
            pltpu.SemaphoreType.DMA,
        ],
        compiler_params=pltpu.CompilerParams(use_tc_tiling_on_sc=False),
    )
    def glove_gather(ctr_hbm, cxt_hbm, tabs, biases,
                     ctr_out, cxt_out, ctr_b_out, cxt_b_out,
                     ctr_idx, cxt_idx, ctr_rows, cxt_rows, ctr_bv, cxt_bv,
                     sem):
        wid = lax.axis_index("s") * _NC + lax.axis_index("c")
        base = wid * b_per_w
        pltpu.sync_copy(ctr_hbm.at[pl.ds(base, b_per_w)], ctr_idx)
        pltpu.sync_copy(cxt_hbm.at[pl.ds(base, b_per_w)], cxt_idx)
        for j in range(b_per_w // _L):
            s = pl.ds(j * _L, _L)
            cxt_idx[s] = cxt_idx[s] + V
        c1 = pltpu.async_copy(tabs.at[ctr_idx], ctr_rows, sem)
        c2 = pltpu.async_copy(tabs.at[cxt_idx], cxt_rows, sem)
        c3 = pltpu.async_copy(biases.at[ctr_idx], ctr_bv, sem)
        c4 = pltpu.async_copy(biases.at[cxt_idx], cxt_bv, sem)
        c1.wait()
        pltpu.sync_copy(ctr_rows, ctr_out.at[pl.ds(base, b_per_w)])
        c2.wait()
        pltpu.sync_copy(cxt_rows, cxt_out.at[pl.ds(base, b_per_w)])
        c3.wait()
        pltpu.sync_copy(ctr_bv, ctr_b_out.at[pl.ds(base, b_per_w)])
        c4.wait()
        pltpu.sync_copy(cxt_bv, cxt_b_out.at[pl.ds(base, b_per_w)])

    return glove_gather


def kernel(ctr, cxt, ctr_table, cxt_table, ctr_bias_table, cxt_bias_table):
    B = ctr.shape[0]
    V, D = ctr_table.shape
    fn = _build(B, D, V)
    tabs = jnp.concatenate([ctr_table, cxt_table], axis=0)
    biases = jnp.concatenate(
        [ctr_bias_table.reshape(-1), cxt_bias_table.reshape(-1)])
    ce, xe, cb, xb = fn(
        ctr.astype(jnp.int32),
        cxt.astype(jnp.int32),
        tabs,
        biases,
    )
    return ce, xe, cb.reshape(B, 1), xb.reshape(B, 1)


# speedup vs baseline: 2.2431x; 1.1890x over previous
import functools

import jax
import jax.numpy as jnp
from jax import lax
from jax.experimental import pallas as pl
from jax.experimental.pallas import tpu as pltpu
from jax.experimental.pallas import tpu_sc as plsc

_NC = 2
_NS = 16
_NW = _NC * _NS


@functools.lru_cache(maxsize=None)
def _build(B, D):
    assert B % (8 * _NW) == 0
    b_per_w = B // _NW
    mesh = plsc.VectorSubcoreMesh(core_axis_name="c", subcore_axis_name="s")

    @functools.partial(
        pl.kernel,
        mesh=mesh,
        out_type=(
            jax.ShapeDtypeStruct((B, D), jnp.float32),
            jax.ShapeDtypeStruct((B, D), jnp.float32),
            jax.ShapeDtypeStruct((B,), jnp.float32),
            jax.ShapeDtypeStruct((B,), jnp.float32),
        ),
        scratch_types=[
            pltpu.VMEM((b_per_w,), jnp.int32),
            pltpu.VMEM((b_per_w,), jnp.int32),
            pltpu.VMEM((b_per_w, D), jnp.float32),
            pltpu.VMEM((b_per_w, D), jnp.float32),
            pltpu.VMEM((b_per_w,), jnp.float32),
            pltpu.VMEM((b_per_w,), jnp.float32),
            pltpu.SemaphoreType.DMA,
        ],
        compiler_params=pltpu.CompilerParams(use_tc_tiling_on_sc=False),
    )
    def glove_gather(ctr_hbm, cxt_hbm, ctr_tab, cxt_tab, ctr_bias, cxt_bias,
                     ctr_out, cxt_out, ctr_b_out, cxt_b_out,
                     ctr_idx, cxt_idx, ctr_rows, cxt_rows, ctr_bv, cxt_bv,
                     sem):
        wid = lax.axis_index("s") * _NC + lax.axis_index("c")
        base = wid * b_per_w
        pltpu.sync_copy(ctr_hbm.at[pl.ds(base, b_per_w)], ctr_idx)
        pltpu.sync_copy(cxt_hbm.at[pl.ds(base, b_per_w)], cxt_idx)
        c1 = pltpu.async_copy(ctr_tab.at[ctr_idx], ctr_rows, sem)
        c2 = pltpu.async_copy(cxt_tab.at[cxt_idx], cxt_rows, sem)
        c3 = pltpu.async_copy(ctr_bias.at[ctr_idx], ctr_bv, sem)
        c4 = pltpu.async_copy(cxt_bias.at[cxt_idx], cxt_bv, sem)
        c1.wait()
        pltpu.sync_copy(ctr_rows, ctr_out.at[pl.ds(base, b_per_w)])
        c2.wait()
        pltpu.sync_copy(cxt_rows, cxt_out.at[pl.ds(base, b_per_w)])
        c3.wait()
        pltpu.sync_copy(ctr_bv, ctr_b_out.at[pl.ds(base, b_per_w)])
        c4.wait()
        pltpu.sync_copy(cxt_bv, cxt_b_out.at[pl.ds(base, b_per_w)])

    return glove_gather


def kernel(ctr, cxt, ctr_table, cxt_table, ctr_bias_table, cxt_bias_table):
    B = ctr.shape[0]
    D = ctr_table.shape[1]
    fn = _build(B, D)
    ce, xe, cb, xb = fn(
        ctr.astype(jnp.int32),
        cxt.astype(jnp.int32),
        ctr_table,
        cxt_table,
        ctr_bias_table.reshape(-1),
        cxt_bias_table.reshape(-1),
    )
    return ce, xe, cb.reshape(B, 1), xb.reshape(B, 1)
